# baseline (speedup 1.0000x reference)
import functools

import jax
import jax.numpy as jnp
from jax import lax
from jax.experimental import pallas as pl
from jax.experimental.pallas import tpu as pltpu
from jax.experimental.pallas import tpu_sc as plsc

INP = 16384
OUT = 1024
NNZ = 268435
NT = 32
RPT = INP // NT
C = 16
BIGC = 1024
PAD = 2 * BIGC + 128
G = OUT // 16
NBUF = 4


def _zero_ref(ref, n):
    z = jnp.zeros((16,), jnp.float32)

    def zb(i, _):
        ref[pl.ds(i * 16, 16)] = z
        return 0

    lax.fori_loop(0, n, zb, 0, unroll=8)


def _body(slin, bal, bend, emb, out, bal_v, bend_v, lin_big, lin_sh,
          idxA, idxB, idxC, idxD, rstA, rstB, rstC, rstD,
          dstA, dstB, dstC, dstD, pstA, pstB, pstC, pstD,
          gbufA, gbufB, gbufC, gbufD, acc, zbuf,
          gsemA, gsemB, gsemC, gsemD, fsem):
    t = lax.axis_index("s") * 2 + lax.axis_index("c")
    row_base = t * RPT
    iota = lax.iota(jnp.int32, 16)

    pltpu.sync_copy(bal, bal_v)
    pltpu.sync_copy(bend, bend_v)
    s0 = bal_v[pl.ds(t, 16)][0]
    e0 = bend_v[pl.ds(t, 16)][0]
    nch = (e0 - s0 + (C - 1)) // C
    nch2 = (nch + 1) // 2

    _zero_ref(acc, NBUF * G)
    _zero_ref(zbuf, 8 * G)

    def zrow(i, _):
        pltpu.sync_copy(
            zbuf,
            out.at[pl.ds(pl.multiple_of((row_base + i * 8) * OUT, 8), 8 * OUT)])
        return 0

    lax.fori_loop(0, RPT // 8, zrow, 0, unroll=False)

    def meta(j, idx_r, rst_r, dst_r, pst_r, gbuf_r, gsem_r,
             prev_lin, prev_c, pbase):
        def loadbig(_):
            pltpu.sync_copy(
                slin.at[pl.ds(pl.multiple_of(s0 + j * C, 8), BIGC)], lin_big)
            return 0

        lax.cond((j & (BIGC // C - 1)) == 0, loadbig, lambda _: 0, 0)
        cur = lin_big[pl.ds((j & (BIGC // C - 1)) * C, 16)]
        lin_sh[pl.ds(0, 16)] = jnp.broadcast_to(prev_lin, (16,))
        lin_sh[pl.ds(1, 16)] = cur
        prv = lin_sh[pl.ds(0, 16)]
        eq = cur == prv
        base = jnp.where(eq, jnp.int32(-(2 ** 29)), iota)
        base = jnp.where((iota == 0) & eq, -prev_c, base)
        lasth = plsc.cummax(base)
        cc = iota - lasth + 1
        rowsv = lax.shift_right_logical(cur, 14)
        colsv = lax.bitwise_and(cur, jnp.int32(INP - 1))
        valid = (rowsv >= row_base) & (rowsv < row_base + RPT)
        cf = cc.astype(jnp.float32)
        ec = jnp.exp(cf)
        d = jnp.where(cc == 1, ec, ec - jnp.exp(cf - 1.0))
        d = jnp.where(valid, d, jnp.float32(0.0))
        pref = plsc.cumsum(d) + jnp.broadcast_to(pbase, (16,))
        idx_r[...] = colsv
        rst_r[pl.ds(0, 16)] = rowsv
        dst_r[pl.ds(0, 16)] = d
        pst_r[pl.ds(0, 16)] = pref
        new_prev_lin = cur[15]
        new_prev_c = cc[15]
        new_pbase = pref[15]
        pltpu.async_copy(emb.at[idx_r], gbuf_r, gsem_r)
        return new_prev_lin, new_prev_c, new_pbase

    def issue_flush(ab, row):
        pltpu.async_copy(
            acc.at[pl.ds(pl.multiple_of(ab * OUT, 8), OUT)],
            out.at[pl.ds(pl.multiple_of(row * OUT, 8), OUT)], fsem)

    def wait_flush_unit():
        pltpu.make_async_copy(
            acc.at[pl.ds(0, OUT)], out.at[pl.ds(0, OUT)], fsem).wait()

    def flush_row(ab, fcnt, cur_row, inv):
        def sc(g, _):
            a = acc[pl.ds(ab * OUT + g * 16, 16)]
            acc[pl.ds(ab * OUT + g * 16, 16)] = a * inv
            return 0

        lax.fori_loop(0, G, sc, 0, unroll=8)
        issue_flush(ab, cur_row)
        fcnt = fcnt + 1
        lax.cond(fcnt >= NBUF, lambda _: wait_flush_unit() or 0,
                 lambda _: 0, 0)
        nab = lax.rem(ab + 1, NBUF)

        def zc(g, _):
            acc[pl.ds(nab * OUT + g * 16, 16)] = jnp.zeros((16,), jnp.float32)
            return 0

        lax.fori_loop(0, G, zc, 0, unroll=8)
        return nab, fcnt

    def accum(rst_r, dst_r, pst_r, gbuf_r, last_pref, cur_row, zbase, ab,
              fcnt):
        def lane(l, carry):
            cur_row, zbase, ab, fcnt = carry
            r = rst_r[pl.ds(l, 16)][0]
            own = (r >= row_base) & (r < row_base + RPT)

            def do(carry):
                cur_row, zbase, ab, fcnt = carry
                send = jnp.where(
                    l == 0, last_pref,
                    pst_r[pl.ds(jnp.maximum(l - 1, 0), 16)][0])

                def changed(args):
                    cur_row, zbase, ab, fcnt = args

                    def wflush(_):
                        bz = (jnp.broadcast_to(send, (16,))
                              - jnp.broadcast_to(zbase, (16,)))
                        inv = jnp.broadcast_to(jnp.float32(1.0), (16,)) / bz
                        return flush_row(ab, fcnt, cur_row, inv)

                    ab, fcnt = lax.cond(cur_row >= 0, wflush,
                                        lambda _: (ab, fcnt), 0)
                    return r, send, ab, fcnt

                cur_row, zbase, ab, fcnt = lax.cond(
                    r != cur_row, changed, lambda a: a,
                    (cur_row, zbase, ab, fcnt))

                dv = jnp.broadcast_to(dst_r[pl.ds(l, 16)][0], (16,))

                def ag(g, _):
                    gv = gbuf_r[l, pl.ds(g * 16, 16)]
                    plsc.addupdate(acc.at[pl.ds(ab * OUT + g * 16, 16)],
                                   dv * gv)
                    return 0

                lax.fori_loop(0, G, ag, 0, unroll=8)
                return cur_row, zbase, ab, fcnt

            carry = lax.cond(own, do, lambda c: c, (cur_row, zbase, ab, fcnt))
            return carry

        cur_row, zbase, ab, fcnt = lax.fori_loop(
            0, 16, lane, (cur_row, zbase, ab, fcnt), unroll=False)
        return pst_r[pl.ds(15, 16)][0], cur_row, zbase, ab, fcnt

    def waitg(idx_r, gbuf_r, gsem_r):
        pltpu.make_async_copy(emb.at[idx_r], gbuf_r, gsem_r).wait()

    bufs = [(idxA, rstA, dstA, pstA, gbufA, gsemA),
            (idxB, rstB, dstB, pstB, gbufB, gsemB),
            (idxC, rstC, dstC, pstC, gbufC, gsemC),
            (idxD, rstD, dstD, pstD, gbufD, gsemD)]
    DEPTH = 4
    nch4 = (nch + DEPTH - 1) // DEPTH

    mcar = (jnp.int32(-1), jnp.int32(0), jnp.float32(0.0))
    for _p in range(DEPTH - 1):
        mcar = meta(jnp.int32(_p), *bufs[_p], *mcar)

    def step(j4, carry):
        mc0, mc1, mc2, last_pref, cur_row, zbase, ab, fcnt = carry
        mcar = (mc0, mc1, mc2)
        acar = (last_pref, cur_row, zbase, ab, fcnt)
        for b in range(DEPTH):
            j = DEPTH * j4 + b
            mcar = meta(j + DEPTH - 1, *bufs[(b + DEPTH - 1) % DEPTH], *mcar)
            waitg(bufs[b][0], bufs[b][4], bufs[b][5])
            acar = accum(bufs[b][1], bufs[b][2], bufs[b][3], bufs[b][4],
                         *acar)
        return (*mcar, *acar)

    carry0 = (*mcar, jnp.float32(0.0), jnp.int32(-1),
              jnp.float32(0.0), jnp.int32(0), jnp.int32(0))
    carry = lax.fori_loop(0, nch4, step, carry0, unroll=False)
    _, _, _, last_pref, cur_row, zbase, ab, fcnt = carry

    for _p in range(DEPTH - 1):
        waitg(bufs[_p][0], bufs[_p][4], bufs[_p][5])

    def final(args):
        abv, fcv = args
        bz = (jnp.broadcast_to(last_pref, (16,))
              - jnp.broadcast_to(zbase, (16,)))
        inv = jnp.broadcast_to(jnp.float32(1.0), (16,)) / bz
        return flush_row(abv, fcv, cur_row, inv)

    ab, fcnt = lax.cond(cur_row >= 0, final, lambda a: a, (ab, fcnt))
    outst = jnp.minimum(fcnt, NBUF - 1)
    for _db in range(NBUF - 1):
        lax.cond(outst > _db, lambda _: wait_flush_unit() or 0,
                 lambda _: 0, 0)


_mesh = plsc.VectorSubcoreMesh(core_axis_name="c", subcore_axis_name="s",
                               num_cores=2, num_subcores=16)

_sc_call = functools.partial(
    pl.kernel,
    out_type=jax.ShapeDtypeStruct((INP * OUT,), jnp.float32),
    mesh=_mesh,
    scratch_types=[
        pltpu.VMEM((48,), jnp.int32),
        pltpu.VMEM((48,), jnp.int32),
        pltpu.VMEM((BIGC,), jnp.int32),
        pltpu.VMEM((24,), jnp.int32),
        pltpu.VMEM((16,), jnp.int32),
        pltpu.VMEM((16,), jnp.int32),
        pltpu.VMEM((16,), jnp.int32),
        pltpu.VMEM((16,), jnp.int32),
        pltpu.VMEM((32,), jnp.int32),
        pltpu.VMEM((32,), jnp.int32),
        pltpu.VMEM((32,), jnp.int32),
        pltpu.VMEM((32,), jnp.int32),
        pltpu.VMEM((32,), jnp.float32),
        pltpu.VMEM((32,), jnp.float32),
        pltpu.VMEM((32,), jnp.float32),
        pltpu.VMEM((32,), jnp.float32),
        pltpu.VMEM((32,), jnp.float32),
        pltpu.VMEM((32,), jnp.float32),
        pltpu.VMEM((32,), jnp.float32),
        pltpu.VMEM((32,), jnp.float32),
        pltpu.VMEM((16, OUT), jnp.float32),
        pltpu.VMEM((16, OUT), jnp.float32),
        pltpu.VMEM((16, OUT), jnp.float32),
        pltpu.VMEM((16, OUT), jnp.float32),
        pltpu.VMEM((NBUF * OUT,), jnp.float32),
        pltpu.VMEM((8 * OUT,), jnp.float32),
        pltpu.SemaphoreType.DMA,
        pltpu.SemaphoreType.DMA,
        pltpu.SemaphoreType.DMA,
        pltpu.SemaphoreType.DMA,
        pltpu.SemaphoreType.DMA,
    ],
    compiler_params=pltpu.CompilerParams(needs_layout_passes=False),
)(_body)


def kernel(indices, emb_weight):
    rows = indices[0].astype(jnp.int32)
    cols = indices[1].astype(jnp.int32)
    lin = rows * INP + cols
    slin = jnp.sort(lin)
    marks = (jnp.arange(NT + 1, dtype=jnp.int32) * (RPT * INP)).astype(jnp.int32)
    b = jnp.searchsorted(slin, marks, side="left").astype(jnp.int32)
    bal = jnp.bitwise_and(b[:NT], jnp.int32(-8))
    bend = b[1:]
    pad16 = jnp.zeros((16,), jnp.int32)
    bal48 = jnp.concatenate([bal, pad16])
    bend48 = jnp.concatenate([bend, pad16])
    slin_p = jnp.concatenate(
        [slin, jnp.full((PAD,), jnp.int32(1 << 28), dtype=jnp.int32)])
    out = _sc_call(slin_p, bal48, bend48, emb_weight)
    return out.reshape(INP, OUT)


# speedup vs baseline: 2.2347x; 2.2347x over previous
import functools

import jax
import jax.numpy as jnp
from jax import lax
from jax.experimental import pallas as pl
from jax.experimental.pallas import tpu as pltpu
from jax.experimental.pallas import tpu_sc as plsc

INP = 16384
OUT = 1024
NNZ = 268435
NT = 32
RPT = INP // NT
C = 16
BIGC = 1024
PAD = 2 * BIGC + 128
G = OUT // 16
NBUF = 4


def _zero_ref(ref, n):
    z = jnp.zeros((16,), jnp.float32)

    @plsc.parallel_loop(0, n, unroll=8)
    def zb(i):
        ref[pl.ds(i * 16, 16)] = z


def _body(slin, bal, bend, emb, out, bal_v, bend_v, lin_big, lin_sh,
          idxA, idxB, idxC, idxD, rstA, rstB, rstC, rstD,
          dstA, dstB, dstC, dstD, pstA, pstB, pstC, pstD,
          gbufA, gbufB, gbufC, gbufD, acc, zbuf,
          gsemA, gsemB, gsemC, gsemD, fsem):
    t = lax.axis_index("s") * 2 + lax.axis_index("c")
    row_base = t * RPT
    iota = lax.iota(jnp.int32, 16)

    pltpu.sync_copy(bal, bal_v)
    pltpu.sync_copy(bend, bend_v)
    s0 = bal_v[pl.ds(t, 16)][0]
    e0 = bend_v[pl.ds(t, 16)][0]
    nch = (e0 - s0 + (C - 1)) // C
    nch2 = (nch + 1) // 2

    _zero_ref(acc, NBUF * G)
    _zero_ref(zbuf, 8 * G)

    def zrow(i, _):
        pltpu.sync_copy(
            zbuf,
            out.at[pl.ds(pl.multiple_of((row_base + i * 8) * OUT, 8), 8 * OUT)])
        return 0

    lax.fori_loop(0, RPT // 8, zrow, 0, unroll=False)

    def meta(j, idx_r, rst_r, dst_r, pst_r, gbuf_r, gsem_r,
             prev_lin, prev_c, pbase):
        def loadbig(_):
            pltpu.sync_copy(
                slin.at[pl.ds(pl.multiple_of(s0 + j * C, 8), BIGC)], lin_big)
            return 0

        lax.cond((j & (BIGC // C - 1)) == 0, loadbig, lambda _: 0, 0)
        cur = lin_big[pl.ds((j & (BIGC // C - 1)) * C, 16)]
        lin_sh[pl.ds(0, 16)] = jnp.broadcast_to(prev_lin, (16,))
        lin_sh[pl.ds(1, 16)] = cur
        prv = lin_sh[pl.ds(0, 16)]
        eq = cur == prv
        base = jnp.where(eq, jnp.int32(-(2 ** 29)), iota)
        base = jnp.where((iota == 0) & eq, -prev_c, base)
        lasth = plsc.cummax(base)
        cc = iota - lasth + 1
        rowsv = lax.shift_right_logical(cur, 14)
        colsv = lax.bitwise_and(cur, jnp.int32(INP - 1))
        valid = (rowsv >= row_base) & (rowsv < row_base + RPT)
        cf = cc.astype(jnp.float32)
        ec = jnp.exp(cf)
        d = jnp.where(cc == 1, ec, ec - jnp.exp(cf - 1.0))
        d = jnp.where(valid, d, jnp.float32(0.0))
        pref = plsc.cumsum(d) + jnp.broadcast_to(pbase, (16,))
        idx_r[...] = colsv
        rst_r[pl.ds(0, 16)] = rowsv
        dst_r[pl.ds(0, 16)] = d
        pst_r[pl.ds(0, 16)] = pref
        new_prev_lin = cur[15]
        new_prev_c = cc[15]
        new_pbase = pref[15]
        pltpu.async_copy(emb.at[idx_r], gbuf_r, gsem_r)
        return new_prev_lin, new_prev_c, new_pbase

    def issue_flush(ab, row):
        pltpu.async_copy(
            acc.at[pl.ds(pl.multiple_of(ab * OUT, 8), OUT)],
            out.at[pl.ds(pl.multiple_of(row * OUT, 8), OUT)], fsem)

    def wait_flush_unit():
        pltpu.make_async_copy(
            acc.at[pl.ds(0, OUT)], out.at[pl.ds(0, OUT)], fsem).wait()

    def flush_row(ab, fcnt, cur_row, inv):
        abase = ab * OUT

        @plsc.parallel_loop(0, G, unroll=8)
        def sc(g):
            a = acc[pl.ds(abase + g * 16, 16)]
            acc[pl.ds(abase + g * 16, 16)] = a * inv

        issue_flush(ab, cur_row)
        fcnt = fcnt + 1
        lax.cond(fcnt >= NBUF, lambda _: wait_flush_unit() or 0,
                 lambda _: 0, 0)
        nab = lax.rem(ab + 1, NBUF)
        nbase = nab * OUT
        zv = jnp.zeros((16,), jnp.float32)

        @plsc.parallel_loop(0, G, unroll=8)
        def zc(g):
            acc[pl.ds(nbase + g * 16, 16)] = zv

        return nab, fcnt

    def accum(rst_r, dst_r, pst_r, gbuf_r, last_pref, cur_row, zbase, ab,
              fcnt):
        def lane(l, carry):
            cur_row, zbase, ab, fcnt = carry
            r = rst_r[pl.ds(l, 16)][0]
            own = (r >= row_base) & (r < row_base + RPT)

            def do(carry):
                cur_row, zbase, ab, fcnt = carry
                send = jnp.where(
                    l == 0, last_pref,
                    pst_r[pl.ds(jnp.maximum(l - 1, 0), 16)][0])

                def changed(args):
                    cur_row, zbase, ab, fcnt = args

                    def wflush(_):
                        bz = (jnp.broadcast_to(send, (16,))
                              - jnp.broadcast_to(zbase, (16,)))
                        inv = jnp.broadcast_to(jnp.float32(1.0), (16,)) / bz
                        return flush_row(ab, fcnt, cur_row, inv)

                    ab, fcnt = lax.cond(cur_row >= 0, wflush,
                                        lambda _: (ab, fcnt), 0)
                    return r, send, ab, fcnt

                cur_row, zbase, ab, fcnt = lax.cond(
                    r != cur_row, changed, lambda a: a,
                    (cur_row, zbase, ab, fcnt))

                dv = jnp.broadcast_to(dst_r[pl.ds(l, 16)][0], (16,))
                abase = ab * OUT

                @plsc.parallel_loop(0, G, unroll=8)
                def ag(g):
                    gv = gbuf_r[l, pl.ds(g * 16, 16)]
                    plsc.addupdate(acc.at[pl.ds(abase + g * 16, 16)],
                                   dv * gv)

                return cur_row, zbase, ab, fcnt

            carry = lax.cond(own, do, lambda c: c, (cur_row, zbase, ab, fcnt))
            return carry

        cur_row, zbase, ab, fcnt = lax.fori_loop(
            0, 16, lane, (cur_row, zbase, ab, fcnt), unroll=False)
        return pst_r[pl.ds(15, 16)][0], cur_row, zbase, ab, fcnt

    def waitg(idx_r, gbuf_r, gsem_r):
        pltpu.make_async_copy(emb.at[idx_r], gbuf_r, gsem_r).wait()

    bufs = [(idxA, rstA, dstA, pstA, gbufA, gsemA),
            (idxB, rstB, dstB, pstB, gbufB, gsemB),
            (idxC, rstC, dstC, pstC, gbufC, gsemC),
            (idxD, rstD, dstD, pstD, gbufD, gsemD)]
    DEPTH = 4
    nch4 = (nch + DEPTH - 1) // DEPTH

    mcar = (jnp.int32(-1), jnp.int32(0), jnp.float32(0.0))
    for _p in range(DEPTH - 1):
        mcar = meta(jnp.int32(_p), *bufs[_p], *mcar)

    def step(j4, carry):
        mc0, mc1, mc2, last_pref, cur_row, zbase, ab, fcnt = carry
        mcar = (mc0, mc1, mc2)
        acar = (last_pref, cur_row, zbase, ab, fcnt)
        for b in range(DEPTH):
            j = DEPTH * j4 + b
            mcar = meta(j + DEPTH - 1, *bufs[(b + DEPTH - 1) % DEPTH], *mcar)
            waitg(bufs[b][0], bufs[b][4], bufs[b][5])
            acar = accum(bufs[b][1], bufs[b][2], bufs[b][3], bufs[b][4],
                         *acar)
        return (*mcar, *acar)

    carry0 = (*mcar, jnp.float32(0.0), jnp.int32(-1),
              jnp.float32(0.0), jnp.int32(0), jnp.int32(0))
    carry = lax.fori_loop(0, nch4, step, carry0, unroll=False)
    _, _, _, last_pref, cur_row, zbase, ab, fcnt = carry

    for _p in range(DEPTH - 1):
        waitg(bufs[_p][0], bufs[_p][4], bufs[_p][5])

    def final(args):
        abv, fcv = args
        bz = (jnp.broadcast_to(last_pref, (16,))
              - jnp.broadcast_to(zbase, (16,)))
        inv = jnp.broadcast_to(jnp.float32(1.0), (16,)) / bz
        return flush_row(abv, fcv, cur_row, inv)

    ab, fcnt = lax.cond(cur_row >= 0, final, lambda a: a, (ab, fcnt))
    outst = jnp.minimum(fcnt, NBUF - 1)
    for _db in range(NBUF - 1):
        lax.cond(outst > _db, lambda _: wait_flush_unit() or 0,
                 lambda _: 0, 0)


_mesh = plsc.VectorSubcoreMesh(core_axis_name="c", subcore_axis_name="s",
                               num_cores=2, num_subcores=16)

_sc_call = functools.partial(
    pl.kernel,
    out_type=jax.ShapeDtypeStruct((INP * OUT,), jnp.float32),
    mesh=_mesh,
    scratch_types=[
        pltpu.VMEM((48,), jnp.int32),
        pltpu.VMEM((48,), jnp.int32),
        pltpu.VMEM((BIGC,), jnp.int32),
        pltpu.VMEM((24,), jnp.int32),
        pltpu.VMEM((16,), jnp.int32),
        pltpu.VMEM((16,), jnp.int32),
        pltpu.VMEM((16,), jnp.int32),
        pltpu.VMEM((16,), jnp.int32),
        pltpu.VMEM((32,), jnp.int32),
        pltpu.VMEM((32,), jnp.int32),
        pltpu.VMEM((32,), jnp.int32),
        pltpu.VMEM((32,), jnp.int32),
        pltpu.VMEM((32,), jnp.float32),
        pltpu.VMEM((32,), jnp.float32),
        pltpu.VMEM((32,), jnp.float32),
        pltpu.VMEM((32,), jnp.float32),
        pltpu.VMEM((32,), jnp.float32),
        pltpu.VMEM((32,), jnp.float32),
        pltpu.VMEM((32,), jnp.float32),
        pltpu.VMEM((32,), jnp.float32),
        pltpu.VMEM((16, OUT), jnp.float32),
        pltpu.VMEM((16, OUT), jnp.float32),
        pltpu.VMEM((16, OUT), jnp.float32),
        pltpu.VMEM((16, OUT), jnp.float32),
        pltpu.VMEM((NBUF * OUT,), jnp.float32),
        pltpu.VMEM((8 * OUT,), jnp.float32),
        pltpu.SemaphoreType.DMA,
        pltpu.SemaphoreType.DMA,
        pltpu.SemaphoreType.DMA,
        pltpu.SemaphoreType.DMA,
        pltpu.SemaphoreType.DMA,
    ],
    compiler_params=pltpu.CompilerParams(needs_layout_passes=False),
)(_body)


def kernel(indices, emb_weight):
    rows = indices[0].astype(jnp.int32)
    cols = indices[1].astype(jnp.int32)
    lin = rows * INP + cols
    slin = jnp.sort(lin)
    marks = (jnp.arange(NT + 1, dtype=jnp.int32) * (RPT * INP)).astype(jnp.int32)
    b = jnp.searchsorted(slin, marks, side="left").astype(jnp.int32)
    bal = jnp.bitwise_and(b[:NT], jnp.int32(-8))
    bend = b[1:]
    pad16 = jnp.zeros((16,), jnp.int32)
    bal48 = jnp.concatenate([bal, pad16])
    bend48 = jnp.concatenate([bend, pad16])
    slin_p = jnp.concatenate(
        [slin, jnp.full((PAD,), jnp.int32(1 << 28), dtype=jnp.int32)])
    out = _sc_call(slin_p, bal48, bend48, emb_weight)
    return out.reshape(INP, OUT)
